# baseline (speedup 1.0000x reference)
import functools

import jax
import jax.numpy as jnp
from jax import lax
from jax.experimental import pallas as pl
from jax.experimental.pallas import tpu as pltpu
from jax.experimental.pallas import tpu_sc as plsc

_BS = 128
_COPY = 64
_NW = 16
_RPW = _BS // _NW
_POOL_LEN = 32768


def _sc_gather(rpi, table, seq, tk1, st1):
    mesh = plsc.VectorSubcoreMesh(
        core_axis_name="c", subcore_axis_name="s", num_cores=1
    )

    @functools.partial(
        pl.kernel,
        out_type=jax.ShapeDtypeStruct((_BS * _COPY,), jnp.int32),
        mesh=mesh,
        compiler_params=pltpu.CompilerParams(
            needs_layout_passes=False, skip_device_barrier=True
        ),
        scratch_types=[
            pltpu.VMEM((_BS + 16,), jnp.int32),
            pltpu.VMEM((_BS + 16,), jnp.int32),
            pltpu.VMEM((16,), jnp.int32),
            pltpu.VMEM((_RPW, 8, 256), jnp.int32),
            pltpu.VMEM((_RPW * _COPY,), jnp.int32),
            pltpu.SemaphoreType.DMA,
            pltpu.SemaphoreType.DMA((_RPW,)),
            pltpu.SemaphoreType.DMA,
        ],
    )
    def k(rpi_hbm, table_hbm, seq_hbm, tk_hbm, st_hbm, out_hbm,
          rpi_v, seq_v, sc_v, buf_v, out_v, sem, gsem, osem):
        wid = lax.axis_index("s") + lax.axis_index("c") * 16
        base = wid * _RPW
        c1 = pltpu.async_copy(rpi_hbm, rpi_v.at[pl.ds(0, _BS)], sem)
        c2 = pltpu.async_copy(seq_hbm, seq_v.at[pl.ds(0, _BS)], sem)
        c3 = pltpu.async_copy(tk_hbm, sc_v.at[pl.ds(0, 1)], sem)
        c4 = pltpu.async_copy(st_hbm, sc_v.at[pl.ds(8, 1)], sem)
        c1.wait()
        c2.wait()
        c3.wait()
        c4.wait()
        scal = sc_v[pl.ds(0, 16)]
        dep = scal[0] * scal[8] - jnp.int32(_COPY)
        rows16 = rpi_v[pl.ds(base, 16)]
        starts16 = seq_v[pl.ds(base, 16)] + dep
        copies = []
        cas = []
        for q in range(_RPW):
            r8 = pl.multiple_of(rows16[q] & ~jnp.int32(7), 8)
            ca = jnp.minimum(
                starts16[q] & ~jnp.int32(127), jnp.int32(_POOL_LEN - 256)
            )
            ca = pl.multiple_of(ca, 128)
            cas.append(ca)
            copies.append(
                pltpu.async_copy(
                    table_hbm.at[pl.ds(r8, 8), pl.ds(ca, 256)],
                    buf_v.at[q],
                    gsem.at[q],
                )
            )
        lanes = lax.iota(jnp.int32, 16)
        qv = jnp.zeros((16,), jnp.int32)
        out_base = wid * (_RPW * _COPY)
        outs = []
        for q in range(_RPW):
            copies[q].wait()
            rv = jnp.full((16,), rows16[q] & jnp.int32(7), jnp.int32)
            phv = jnp.full((16,), starts16[q] - cas[q], jnp.int32) + lanes
            for j0 in range(0, _COPY, 16):
                lc = phv + j0
                out_v[pl.ds(q * _COPY + j0, 16)] = plsc.load_gather(
                    buf_v, [qv + q, rv, lc]
                )
            outs.append(
                pltpu.async_copy(
                    out_v.at[pl.ds(q * _COPY, _COPY)],
                    out_hbm.at[pl.ds(out_base + q * _COPY, _COPY)],
                    osem,
                )
            )
        for o in outs:
            o.wait()

    return k(rpi, table, seq, tk1, st1)


def kernel(req_pool_indices, req_to_token, seq_lens, topk, speculative_num_steps):
    tk1 = jnp.reshape(jnp.asarray(topk, jnp.int32), (1,))
    st1 = jnp.reshape(jnp.asarray(speculative_num_steps, jnp.int32), (1,))
    return _sc_gather(
        req_pool_indices.astype(jnp.int32),
        req_to_token.astype(jnp.int32),
        seq_lens.astype(jnp.int32),
        tk1,
        st1,
    )


# speedup vs baseline: 1.0070x; 1.0070x over previous
import functools

import jax
import jax.numpy as jnp
from jax import lax
from jax.experimental import pallas as pl
from jax.experimental.pallas import tpu as pltpu
from jax.experimental.pallas import tpu_sc as plsc

_BS = 128
_COPY = 64
_NW = 16
_RPW = _BS // _NW
_POOL_LEN = 32768


def _sc_gather(rpi, table, seq, tk1, st1):
    mesh = plsc.VectorSubcoreMesh(
        core_axis_name="c", subcore_axis_name="s", num_cores=1
    )

    @functools.partial(
        pl.kernel,
        out_type=jax.ShapeDtypeStruct((_BS * _COPY,), jnp.int32),
        mesh=mesh,
        compiler_params=pltpu.CompilerParams(
            needs_layout_passes=False, skip_device_barrier=True
        ),
        scratch_types=[
            pltpu.VMEM((_BS + 16,), jnp.int32),
            pltpu.VMEM((_BS + 16,), jnp.int32),
            pltpu.VMEM((16,), jnp.int32),
            pltpu.VMEM((_RPW, 2, 8, 128), jnp.int32),
            pltpu.VMEM((_RPW * _COPY,), jnp.int32),
            pltpu.SemaphoreType.DMA,
            pltpu.SemaphoreType.DMA((_RPW,)),
            pltpu.SemaphoreType.DMA,
        ],
    )
    def k(rpi_hbm, table_hbm, seq_hbm, tk_hbm, st_hbm, out_hbm,
          rpi_v, seq_v, sc_v, buf_v, out_v, sem, gsem, osem):
        wid = lax.axis_index("s") + lax.axis_index("c") * 16
        base = wid * _RPW
        c1 = pltpu.async_copy(rpi_hbm, rpi_v.at[pl.ds(0, _BS)], sem)
        c2 = pltpu.async_copy(seq_hbm, seq_v.at[pl.ds(0, _BS)], sem)
        c3 = pltpu.async_copy(tk_hbm, sc_v.at[pl.ds(0, 1)], sem)
        c4 = pltpu.async_copy(st_hbm, sc_v.at[pl.ds(8, 1)], sem)
        c1.wait()
        c2.wait()
        c3.wait()
        c4.wait()
        scal = sc_v[pl.ds(0, 16)]
        dep = scal[0] * scal[8] - jnp.int32(_COPY)
        rows16 = rpi_v[pl.ds(base, 16)]
        starts16 = seq_v[pl.ds(base, 16)] + dep
        copies = []
        tile2 = []
        need2s = []
        for q in range(_RPW):
            r8 = pl.multiple_of(rows16[q] & ~jnp.int32(7), 8)
            ph = starts16[q] & jnp.int32(127)
            ca = pl.multiple_of(starts16[q] - ph, 128)
            copies.append(
                pltpu.async_copy(
                    table_hbm.at[pl.ds(r8, 8), pl.ds(ca, 128)],
                    buf_v.at[q, 0],
                    gsem.at[q],
                )
            )
            need2 = ph > jnp.int32(128 - _COPY)
            need2s.append(need2)
            t2 = pltpu.make_async_copy(
                table_hbm.at[pl.ds(r8, 8), pl.ds(ca + 128, 128)],
                buf_v.at[q, 1],
                gsem.at[q],
            )
            tile2.append(t2)
            pl.when(need2)(t2.start)
        lanes = lax.iota(jnp.int32, 16)
        qv = jnp.zeros((16,), jnp.int32)
        out_base = wid * (_RPW * _COPY)
        outs = []
        for q in range(_RPW):
            copies[q].wait()
            pl.when(need2s[q])(tile2[q].wait)
            rv = jnp.full((16,), rows16[q] & jnp.int32(7), jnp.int32)
            phv = jnp.full((16,), starts16[q] & jnp.int32(127), jnp.int32) + lanes
            for j0 in range(0, _COPY, 16):
                lc = phv + j0
                out_v[pl.ds(q * _COPY + j0, 16)] = plsc.load_gather(
                    buf_v, [qv + q, lc >> 7, rv, lc & jnp.int32(127)]
                )
            outs.append(
                pltpu.async_copy(
                    out_v.at[pl.ds(q * _COPY, _COPY)],
                    out_hbm.at[pl.ds(out_base + q * _COPY, _COPY)],
                    osem,
                )
            )
        for o in outs:
            o.wait()

    return k(rpi, table, seq, tk1, st1)


def kernel(req_pool_indices, req_to_token, seq_lens, topk, speculative_num_steps):
    tk1 = jnp.reshape(jnp.asarray(topk, jnp.int32), (1,))
    st1 = jnp.reshape(jnp.asarray(speculative_num_steps, jnp.int32), (1,))
    return _sc_gather(
        req_pool_indices.astype(jnp.int32),
        req_to_token.astype(jnp.int32),
        seq_lens.astype(jnp.int32),
        tk1,
        st1,
    )
